# baseline (speedup 1.0000x reference)
import functools

import jax
import jax.numpy as jnp
from jax import lax
from jax.experimental import pallas as pl
from jax.experimental.pallas import tpu as pltpu
from jax.experimental.pallas import tpu_sc as plsc

VOCAB = 100000
EMB = 128
OUT = 256
NTOK = 1024 * 200

NSEG = 4
SEG = NTOK // NSEG

NC = 2
NS = 16
NW = NC * NS
BPW = SEG // NW
CHUNK = 64
NCHUNK = BPW // CHUNK

RB = 6400
NBLK = SEG // RB


def _gather_body(tab_hbm, idx_hbm, out_hbm, idx_v, buf_v, gsem):
    wid = lax.axis_index("s") * NC + lax.axis_index("c")
    base = wid * BPW
    pltpu.sync_copy(idx_hbm.at[pl.ds(base, BPW)], idx_v)

    def chunk(j, carry):
        off = j * CHUNK
        pltpu.async_copy(
            tab_hbm.at[idx_v.at[pl.ds(off, CHUNK)]], buf_v, gsem
        ).wait()
        pltpu.sync_copy(buf_v, out_hbm.at[pl.ds(base + off, CHUNK)])
        return carry

    lax.fori_loop(0, NCHUNK, chunk, 0)


def _sc_gather(emb_table, idx_seg):
    mesh = plsc.VectorSubcoreMesh(core_axis_name="c", subcore_axis_name="s")
    f = pl.kernel(
        _gather_body,
        out_type=jax.ShapeDtypeStruct((SEG, EMB), jnp.float32),
        mesh=mesh,
        scratch_types=[
            pltpu.VMEM((BPW,), jnp.int32),
            pltpu.VMEM((CHUNK, EMB), jnp.float32),
            pltpu.SemaphoreType.DMA,
        ],
    )
    return f(emb_table, idx_seg)


def _mm_body(x_ref, w_ref, b_ref, o_ref):
    o_ref[...] = jnp.maximum(
        jnp.dot(x_ref[...], w_ref[...], preferred_element_type=jnp.float32)
        + b_ref[...],
        0.0,
    )


def _mm_body_alias(x_ref, w_ref, b_ref, y_ref, o_ref):
    _mm_body(x_ref, w_ref, b_ref, o_ref)


def _tc_matmul_seg(x_s, W, b2d, s, y=None):
    out_map = lambda i, s=s: (s * NBLK + i, 0)
    in_specs = [
        pl.BlockSpec((RB, EMB), lambda i: (i, 0)),
        pl.BlockSpec((EMB, OUT), lambda i: (0, 0)),
        pl.BlockSpec((1, OUT), lambda i: (0, 0)),
    ]
    args = (x_s, W, b2d)
    body = _mm_body
    aliases = {}
    if y is not None:
        in_specs.append(pl.BlockSpec(memory_space=pl.ANY))
        args = (x_s, W, b2d, y)
        body = _mm_body_alias
        aliases = {3: 0}
    return pl.pallas_call(
        body,
        grid=(NBLK,),
        in_specs=in_specs,
        out_specs=pl.BlockSpec((RB, OUT), out_map),
        out_shape=jax.ShapeDtypeStruct((NTOK, OUT), jnp.float32),
        input_output_aliases=aliases,
    )(*args)


def kernel(text, emb_table, W, b):
    B, L = text.shape
    idx = text.reshape(-1).astype(jnp.int32)
    b2d = b.reshape(1, OUT)
    y = None
    for s in range(NSEG):
        x_s = _sc_gather(emb_table, idx[s * SEG:(s + 1) * SEG])
        y = _tc_matmul_seg(x_s, W, b2d, s, y)
    return y.reshape(B, L, OUT)


# speedup vs baseline: 1.2810x; 1.2810x over previous
import functools

import jax
import jax.numpy as jnp
from jax import lax
from jax.experimental import pallas as pl
from jax.experimental.pallas import tpu as pltpu
from jax.experimental.pallas import tpu_sc as plsc

VOCAB = 100000
EMB = 128
OUT = 256
NTOK = 1024 * 200

NSEG = 2
SEG = NTOK // NSEG

NC = 2
NS = 16
NW = NC * NS
BPW = SEG // NW
CHUNK = 80
NCHUNK = BPW // CHUNK
NBUF = 4
NROUND = NCHUNK // NBUF

RB = 6400
NBLK = SEG // RB


def _gather_body(tab_hbm, idx_hbm, out_hbm, idx_v, buf_v, *sems):
    gsem, osem = sems[:NBUF], sems[NBUF:]
    wid = lax.axis_index("s") * NC + lax.axis_index("c")
    base = wid * BPW
    pltpu.sync_copy(idx_hbm.at[pl.ds(base, BPW)], idx_v)

    def gstart(j, k):
        pltpu.async_copy(
            tab_hbm.at[idx_v.at[pl.ds(j * CHUNK, CHUNK)]], buf_v.at[k], gsem[k]
        )

    def gwait(k):
        pltpu.make_async_copy(
            tab_hbm.at[idx_v.at[pl.ds(0, CHUNK)]], buf_v.at[k], gsem[k]
        ).wait()

    def ostart(j, k):
        pltpu.async_copy(
            buf_v.at[k], out_hbm.at[pl.ds(base + j * CHUNK, CHUNK)], osem[k]
        )

    def owait(k):
        pltpu.make_async_copy(
            buf_v.at[k], out_hbm.at[pl.ds(base, CHUNK)], osem[k]
        ).wait()

    for k in range(NBUF):
        gstart(k, k)

    def round_body(it, carry):
        for k in range(NBUF):
            j = it * NBUF + k
            gwait(k)
            ostart(j, k)
            owait(k)
            gstart(j + NBUF, k)
        return carry

    lax.fori_loop(0, NROUND - 1, round_body, 0)

    for k in range(NBUF):
        j = (NROUND - 1) * NBUF + k
        gwait(k)
        ostart(j, k)
    for k in range(NBUF):
        owait(k)


def _sc_gather(emb_table, idx_seg):
    mesh = plsc.VectorSubcoreMesh(core_axis_name="c", subcore_axis_name="s")
    f = pl.kernel(
        _gather_body,
        out_type=jax.ShapeDtypeStruct((SEG, EMB), jnp.float32),
        mesh=mesh,
        scratch_types=[
            pltpu.VMEM((BPW,), jnp.int32),
            pltpu.VMEM((NBUF, CHUNK, EMB), jnp.float32),
        ] + [pltpu.SemaphoreType.DMA] * (2 * NBUF),
    )
    return f(emb_table, idx_seg)


def _mm_body(x_ref, w_ref, b_ref, o_ref):
    o_ref[...] = jnp.maximum(
        jnp.dot(x_ref[...], w_ref[...], preferred_element_type=jnp.float32)
        + b_ref[...],
        0.0,
    )


def _mm_body_alias(x_ref, w_ref, b_ref, y_ref, o_ref):
    _mm_body(x_ref, w_ref, b_ref, o_ref)


def _tc_matmul_seg(x_s, W, b2d, s, y=None):
    out_map = lambda i, s=s: (s * NBLK + i, 0)
    in_specs = [
        pl.BlockSpec((RB, EMB), lambda i: (i, 0)),
        pl.BlockSpec((EMB, OUT), lambda i: (0, 0)),
        pl.BlockSpec((1, OUT), lambda i: (0, 0)),
    ]
    args = (x_s, W, b2d)
    body = _mm_body
    aliases = {}
    if y is not None:
        in_specs.append(pl.BlockSpec(memory_space=pl.ANY))
        args = (x_s, W, b2d, y)
        body = _mm_body_alias
        aliases = {3: 0}
    return pl.pallas_call(
        body,
        grid=(NBLK,),
        in_specs=in_specs,
        out_specs=pl.BlockSpec((RB, OUT), out_map),
        out_shape=jax.ShapeDtypeStruct((NTOK, OUT), jnp.float32),
        input_output_aliases=aliases,
    )(*args)


def kernel(text, emb_table, W, b):
    B, L = text.shape
    idx = text.reshape(-1).astype(jnp.int32)
    b2d = b.reshape(1, OUT)
    y = None
    for s in range(NSEG):
        x_s = _sc_gather(emb_table, idx[s * SEG:(s + 1) * SEG])
        y = _tc_matmul_seg(x_s, W, b2d, s, y)
    return y.reshape(B, L, OUT)


# speedup vs baseline: 1.3112x; 1.0236x over previous
import functools

import jax
import jax.numpy as jnp
from jax import lax
from jax.experimental import pallas as pl
from jax.experimental.pallas import tpu as pltpu
from jax.experimental.pallas import tpu_sc as plsc

VOCAB = 100000
EMB = 128
OUT = 256
NTOK = 1024 * 200

NSEG = 2
SEG = NTOK // NSEG

NC = 2
NS = 16
NW = NC * NS
BPW = SEG // NW
CHUNK = 80
NCHUNK = BPW // CHUNK
NBUF = 4
NROUND = NCHUNK // NBUF

RB = 12800
NBLK = SEG // RB


def _gather_body(tab_hbm, idx_hbm, out_hbm, idx_v, buf_v, *sems):
    gsem, osem = sems[:NBUF], sems[NBUF:]
    wid = lax.axis_index("s") * NC + lax.axis_index("c")
    base = wid * BPW
    pltpu.sync_copy(idx_hbm.at[pl.ds(base, BPW)], idx_v)

    def gstart(j, k):
        pltpu.async_copy(
            tab_hbm.at[idx_v.at[pl.ds(j * CHUNK, CHUNK)]], buf_v.at[k], gsem[k]
        )

    def gwait(k):
        pltpu.make_async_copy(
            tab_hbm.at[idx_v.at[pl.ds(0, CHUNK)]], buf_v.at[k], gsem[k]
        ).wait()

    def ostart(j, k):
        pltpu.async_copy(
            buf_v.at[k], out_hbm.at[pl.ds(base + j * CHUNK, CHUNK)], osem[k]
        )

    def owait(k):
        pltpu.make_async_copy(
            buf_v.at[k], out_hbm.at[pl.ds(base, CHUNK)], osem[k]
        ).wait()

    for k in range(NBUF):
        gstart(k, k)

    def round_body(it, carry):
        for k in range(NBUF):
            j = it * NBUF + k
            gwait(k)
            ostart(j, k)
            owait(k)
            gstart(j + NBUF, k)
        return carry

    lax.fori_loop(0, NROUND - 1, round_body, 0)

    for k in range(NBUF):
        j = (NROUND - 1) * NBUF + k
        gwait(k)
        ostart(j, k)
    for k in range(NBUF):
        owait(k)


def _sc_gather(emb_table, idx_seg):
    mesh = plsc.VectorSubcoreMesh(core_axis_name="c", subcore_axis_name="s")
    f = pl.kernel(
        _gather_body,
        out_type=jax.ShapeDtypeStruct((SEG, EMB), jnp.float32),
        mesh=mesh,
        scratch_types=[
            pltpu.VMEM((BPW,), jnp.int32),
            pltpu.VMEM((NBUF, CHUNK, EMB), jnp.float32),
        ] + [pltpu.SemaphoreType.DMA] * (2 * NBUF),
    )
    return f(emb_table, idx_seg)


def _mm_body(x_ref, w_ref, b_ref, o_ref):
    o_ref[...] = jnp.maximum(
        jnp.dot(x_ref[...], w_ref[...], preferred_element_type=jnp.float32)
        + b_ref[...],
        0.0,
    )


def _mm_body_alias(x_ref, w_ref, b_ref, y_ref, o_ref):
    _mm_body(x_ref, w_ref, b_ref, o_ref)


def _tc_matmul_seg(x_s, W, b2d, s, y=None):
    out_map = lambda i, s=s: (s * NBLK + i, 0)
    in_specs = [
        pl.BlockSpec((RB, EMB), lambda i: (i, 0)),
        pl.BlockSpec((EMB, OUT), lambda i: (0, 0)),
        pl.BlockSpec((1, OUT), lambda i: (0, 0)),
    ]
    args = (x_s, W, b2d)
    body = _mm_body
    aliases = {}
    if y is not None:
        in_specs.append(pl.BlockSpec(memory_space=pl.ANY))
        args = (x_s, W, b2d, y)
        body = _mm_body_alias
        aliases = {3: 0}
    return pl.pallas_call(
        body,
        grid=(NBLK,),
        in_specs=in_specs,
        out_specs=pl.BlockSpec((RB, OUT), out_map),
        out_shape=jax.ShapeDtypeStruct((NTOK, OUT), jnp.float32),
        input_output_aliases=aliases,
    )(*args)


def kernel(text, emb_table, W, b):
    B, L = text.shape
    idx = text.reshape(-1).astype(jnp.int32)
    b2d = b.reshape(1, OUT)
    y = None
    for s in range(NSEG):
        x_s = _sc_gather(emb_table, idx[s * SEG:(s + 1) * SEG])
        y = _tc_matmul_seg(x_s, W, b2d, s, y)
    return y.reshape(B, L, OUT)


# speedup vs baseline: 1.3184x; 1.0054x over previous
import functools

import jax
import jax.numpy as jnp
from jax import lax
from jax.experimental import pallas as pl
from jax.experimental.pallas import tpu as pltpu
from jax.experimental.pallas import tpu_sc as plsc

VOCAB = 100000
EMB = 128
OUT = 256
NTOK = 1024 * 200

NSEG = 4
SEG = NTOK // NSEG

NC = 2
NS = 16
NW = NC * NS
BPW = SEG // NW
CHUNK = 80
NCHUNK = BPW // CHUNK
NBUF = 4
NROUND = NCHUNK // NBUF

RB = 12800
NBLK = SEG // RB


def _gather_body(tab_hbm, idx_hbm, out_hbm, idx_v, buf_v, *sems):
    gsem, osem = sems[:NBUF], sems[NBUF:]
    wid = lax.axis_index("s") * NC + lax.axis_index("c")
    base = wid * BPW
    pltpu.sync_copy(idx_hbm.at[pl.ds(base, BPW)], idx_v)

    def gstart(j, k):
        pltpu.async_copy(
            tab_hbm.at[idx_v.at[pl.ds(j * CHUNK, CHUNK)]], buf_v.at[k], gsem[k]
        )

    def gwait(k):
        pltpu.make_async_copy(
            tab_hbm.at[idx_v.at[pl.ds(0, CHUNK)]], buf_v.at[k], gsem[k]
        ).wait()

    def ostart(j, k):
        pltpu.async_copy(
            buf_v.at[k], out_hbm.at[pl.ds(base + j * CHUNK, CHUNK)], osem[k]
        )

    def owait(k):
        pltpu.make_async_copy(
            buf_v.at[k], out_hbm.at[pl.ds(base, CHUNK)], osem[k]
        ).wait()

    for k in range(NBUF):
        gstart(k, k)

    def round_body(it, carry):
        for k in range(NBUF):
            j = it * NBUF + k
            gwait(k)
            ostart(j, k)
            owait(k)
            gstart(j + NBUF, k)
        return carry

    lax.fori_loop(0, NROUND - 1, round_body, 0)

    for k in range(NBUF):
        j = (NROUND - 1) * NBUF + k
        gwait(k)
        ostart(j, k)
    for k in range(NBUF):
        owait(k)


def _sc_gather(emb_table, idx_seg):
    mesh = plsc.VectorSubcoreMesh(core_axis_name="c", subcore_axis_name="s")
    f = pl.kernel(
        _gather_body,
        out_type=jax.ShapeDtypeStruct((SEG, EMB), jnp.float32),
        mesh=mesh,
        scratch_types=[
            pltpu.VMEM((BPW,), jnp.int32),
            pltpu.VMEM((NBUF, CHUNK, EMB), jnp.float32),
        ] + [pltpu.SemaphoreType.DMA] * (2 * NBUF),
    )
    return f(emb_table, idx_seg)


def _mm_body(x_ref, w_ref, b_ref, o_ref):
    o_ref[...] = jnp.maximum(
        jnp.dot(x_ref[...], w_ref[...], preferred_element_type=jnp.float32)
        + b_ref[...],
        0.0,
    )


def _mm_body_alias(x_ref, w_ref, b_ref, y_ref, o_ref):
    _mm_body(x_ref, w_ref, b_ref, o_ref)


def _tc_matmul_seg(x_s, W, b2d, s, y=None):
    out_map = lambda i, s=s: (s * NBLK + i, 0)
    in_specs = [
        pl.BlockSpec((RB, EMB), lambda i: (i, 0)),
        pl.BlockSpec((EMB, OUT), lambda i: (0, 0)),
        pl.BlockSpec((1, OUT), lambda i: (0, 0)),
    ]
    args = (x_s, W, b2d)
    body = _mm_body
    aliases = {}
    if y is not None:
        in_specs.append(pl.BlockSpec(memory_space=pl.ANY))
        args = (x_s, W, b2d, y)
        body = _mm_body_alias
        aliases = {3: 0}
    return pl.pallas_call(
        body,
        grid=(NBLK,),
        in_specs=in_specs,
        out_specs=pl.BlockSpec((RB, OUT), out_map),
        out_shape=jax.ShapeDtypeStruct((NTOK, OUT), jnp.float32),
        input_output_aliases=aliases,
    )(*args)


def kernel(text, emb_table, W, b):
    B, L = text.shape
    idx = text.reshape(-1).astype(jnp.int32)
    b2d = b.reshape(1, OUT)
    y = None
    for s in range(NSEG):
        x_s = _sc_gather(emb_table, idx[s * SEG:(s + 1) * SEG])
        y = _tc_matmul_seg(x_s, W, b2d, s, y)
    return y.reshape(B, L, OUT)
